# Initial kernel scaffold; baseline (speedup 1.0000x reference)
import functools

import jax
import jax.numpy as jnp
from jax import lax
from jax.experimental import pallas as pl
from jax.experimental.pallas import tpu as pltpu
from jax.experimental.pallas import tpu_sc as plsc

D = 128
C = 128
NC, NS = 2, 16
NW = NC * NS
BN = 1024


def _prep_body(f_ref, w_ref, as_ref, ad_ref, x_ref, asrc_ref, adst_ref):
    x = jnp.dot(f_ref[...], w_ref[...], precision=lax.Precision.HIGHEST,
                preferred_element_type=jnp.float32)
    x_ref[...] = x
    asrc_ref[...] = jnp.sum(x * as_ref[...], axis=1, keepdims=True)
    adst_ref[...] = jnp.sum(x * ad_ref[...], axis=1, keepdims=True)


def _tc_prep(feature_p, W, att_src, att_dst, n_pad):
    return pl.pallas_call(
        _prep_body,
        grid=(n_pad // BN,),
        in_specs=[
            pl.BlockSpec((BN, D), lambda i: (i, 0)),
            pl.BlockSpec((D, D), lambda i: (0, 0)),
            pl.BlockSpec((1, D), lambda i: (0, 0)),
            pl.BlockSpec((1, D), lambda i: (0, 0)),
        ],
        out_specs=[
            pl.BlockSpec((BN, D), lambda i: (i, 0)),
            pl.BlockSpec((BN, 1), lambda i: (i, 0)),
            pl.BlockSpec((BN, 1), lambda i: (i, 0)),
        ],
        out_shape=[
            jax.ShapeDtypeStruct((n_pad, D), jnp.float32),
            jax.ShapeDtypeStruct((n_pad, 1), jnp.float32),
            jax.ShapeDtypeStruct((n_pad, 1), jnp.float32),
        ],
    )(feature_p, W, att_src, att_dst)


def _sc_edge_kernel(n_pad, nchunk):
    stripe = n_pad // NS
    mesh = plsc.VectorSubcoreMesh(core_axis_name="c", subcore_axis_name="s")

    @functools.partial(
        pl.kernel,
        mesh=mesh,
        out_type=[
            jax.ShapeDtypeStruct((NC, n_pad, D), jnp.float32),
            jax.ShapeDtypeStruct((NC, n_pad, 16), jnp.float32),
        ],
        scratch_types=[
            pltpu.VMEM((n_pad,), jnp.float32),
            pltpu.VMEM((n_pad,), jnp.float32),
            pltpu.VMEM((nchunk, C), jnp.int32),
            pltpu.VMEM((nchunk, C), jnp.int32),
            pltpu.VMEM((C,), jnp.float32),
            pltpu.VMEM((C, D), jnp.float32),
            pltpu.VMEM((C, 16), jnp.float32),
            pltpu.VMEM_SHARED((n_pad, D), jnp.float32),
            pltpu.VMEM_SHARED((n_pad, 16), jnp.float32),
            pltpu.SemaphoreType.DMA,
        ],
    )
    def sc_edge(x_hbm, asrc_hbm, adst_hbm, src_hbm, dst_hbm,
                acc_hbm, den_hbm,
                asrc_v, adst_v, sidx_v, didx_v, w_v, rows_v, wrow_v,
                acc_sh, den_sh, sem):
        c = lax.axis_index("c")
        s = lax.axis_index("s")
        wid = c * NS + s
        zero16 = jnp.zeros((16,), jnp.float32)

        @pl.loop(0, C)
        def _(r):
            for k in range(D // 16):
                rows_v[r, pl.ds(k * 16, 16)] = zero16
            wrow_v[r, pl.ds(0, 16)] = zero16

        for b in range(stripe // C):
            pltpu.sync_copy(rows_v, acc_sh.at[pl.ds(s * stripe + b * C, C)])
            pltpu.sync_copy(wrow_v, den_sh.at[pl.ds(s * stripe + b * C, C)])

        pltpu.sync_copy(src_hbm.at[wid], sidx_v)
        pltpu.sync_copy(dst_hbm.at[wid], didx_v)
        pltpu.sync_copy(asrc_hbm, asrc_v)
        pltpu.sync_copy(adst_hbm, adst_v)

        plsc.subcore_barrier()

        @pl.loop(0, nchunk)
        def _(i):
            gat = pltpu.async_copy(x_hbm.at[sidx_v.at[i]], rows_v, sem)

            @pl.loop(0, C, step=16)
            def _(j):
                s16 = sidx_v[i, pl.ds(j, 16)]
                d16 = didx_v[i, pl.ds(j, 16)]
                al = (plsc.load_gather(asrc_v, [s16])
                      + plsc.load_gather(adst_v, [d16]))
                al = jnp.where(al >= 0.0, al, al * 0.2)
                w_v[pl.ds(j, 16)] = jnp.exp(al)

            gat.wait()

            @pl.loop(0, C, step=4)
            def _(r0):
                for drr in range(4):
                    r = r0 + drr
                    bc = plsc.load_gather(w_v, [jnp.zeros((16,), jnp.int32) + r])
                    wrow_v[r, pl.ds(0, 16)] = bc
                    for k in range(D // 16):
                        rows_v[r, pl.ds(k * 16, 16)] = (
                            rows_v[r, pl.ds(k * 16, 16)] * bc)

            pltpu.sync_copy(rows_v, acc_sh.at[didx_v.at[i]], add=True)
            pltpu.sync_copy(wrow_v, den_sh.at[didx_v.at[i]], add=True)

        plsc.subcore_barrier()

        for b in range(stripe // C):
            pltpu.sync_copy(acc_sh.at[pl.ds(s * stripe + b * C, C)],
                            acc_hbm.at[c, pl.ds(s * stripe + b * C, C)])
            pltpu.sync_copy(den_sh.at[pl.ds(s * stripe + b * C, C)],
                            den_hbm.at[c, pl.ds(s * stripe + b * C, C)])

    return sc_edge


def _fin_body(acc_ref, den_ref, x_ref, asrc_ref, adst_ref, b_ref, out_ref):
    num = acc_ref[0] + acc_ref[1]
    dtot = den_ref[0, :, 0:1] + den_ref[1, :, 0:1]
    a = asrc_ref[...] + adst_ref[...]
    a = jnp.where(a >= 0.0, a, a * 0.2)
    ws = jnp.exp(a)
    num = num + ws * x_ref[...]
    out_ref[...] = num / (dtot + ws + 1e-16) + b_ref[...]


def _tc_finalize(acc, den, x, asrc, adst, bias2d, n_pad):
    return pl.pallas_call(
        _fin_body,
        grid=(n_pad // BN,),
        in_specs=[
            pl.BlockSpec((NC, BN, D), lambda i: (0, i, 0)),
            pl.BlockSpec((NC, BN, 16), lambda i: (0, i, 0)),
            pl.BlockSpec((BN, D), lambda i: (i, 0)),
            pl.BlockSpec((BN, 1), lambda i: (i, 0)),
            pl.BlockSpec((BN, 1), lambda i: (i, 0)),
            pl.BlockSpec((1, D), lambda i: (0, 0)),
        ],
        out_specs=pl.BlockSpec((BN, D), lambda i: (i, 0)),
        out_shape=jax.ShapeDtypeStruct((n_pad, D), jnp.float32),
    )(acc, den, x, asrc, adst, bias2d)


def kernel(feature, edge_index, W, att_src, att_dst, bias):
    N, d_in = feature.shape
    E = edge_index.shape[1]

    n_pad = ((N + 2047) // 2048) * 2048
    if n_pad == N:
        n_pad += 2048
    e_pad = ((E + NW * C - 1) // (NW * C)) * (NW * C)
    nchunk = e_pad // (NW * C)

    feature_p = jnp.concatenate(
        [feature, jnp.zeros((n_pad - N, d_in), jnp.float32)])
    src = jnp.concatenate([edge_index[0].astype(jnp.int32),
                           jnp.zeros((e_pad - E,), jnp.int32)])
    dst = jnp.concatenate([edge_index[1].astype(jnp.int32),
                           jnp.full((e_pad - E,), N, jnp.int32)])
    src = src.reshape(NW, nchunk, C)
    dst = dst.reshape(NW, nchunk, C)

    x, asrc, adst = _tc_prep(feature_p, W, att_src, att_dst, n_pad)

    acc, den = _sc_edge_kernel(n_pad, nchunk)(
        x, asrc.reshape(n_pad), adst.reshape(n_pad), src, dst)

    out = _tc_finalize(acc, den, x, asrc, adst,
                       bias.reshape(1, D), n_pad)
    return out[:N]


# speedup vs baseline: 21.7383x; 21.7383x over previous
import dataclasses
import functools

import jax
import jax.numpy as jnp
from jax import lax
from jax.experimental import pallas as pl
from jax.experimental.pallas import tpu as pltpu
from jax.experimental.pallas import tpu_sc as plsc

D = 128
C = 128
NC, NS = 2, 16
NW = NC * NS
BN = 1024


def _prep_body(f_ref, w_ref, as_ref, ad_ref, x_ref, asrc_ref, adst_ref):
    x = jnp.dot(f_ref[...], w_ref[...], precision=lax.Precision.HIGHEST,
                preferred_element_type=jnp.float32)
    x_ref[...] = x
    asrc_ref[...] = jnp.sum(x * as_ref[...], axis=1, keepdims=True)
    adst_ref[...] = jnp.sum(x * ad_ref[...], axis=1, keepdims=True)


def _tc_prep(feature_p, W, att_src, att_dst, n_pad):
    return pl.pallas_call(
        _prep_body,
        grid=(n_pad // BN,),
        in_specs=[
            pl.BlockSpec((BN, D), lambda i: (i, 0)),
            pl.BlockSpec((D, D), lambda i: (0, 0)),
            pl.BlockSpec((1, D), lambda i: (0, 0)),
            pl.BlockSpec((1, D), lambda i: (0, 0)),
        ],
        out_specs=[
            pl.BlockSpec((BN, D), lambda i: (i, 0)),
            pl.BlockSpec((BN, 1), lambda i: (i, 0)),
            pl.BlockSpec((BN, 1), lambda i: (i, 0)),
        ],
        out_shape=[
            jax.ShapeDtypeStruct((n_pad, D), jnp.float32),
            jax.ShapeDtypeStruct((n_pad, 1), jnp.float32),
            jax.ShapeDtypeStruct((n_pad, 1), jnp.float32),
        ],
    )(feature_p, W, att_src, att_dst)


def _sc_compiler_params():
    cp = pltpu.CompilerParams()
    if "needs_layout_passes" in pltpu.CompilerParams.__dataclass_fields__:
        cp = dataclasses.replace(cp, needs_layout_passes=False)
    return cp


def _sc_weights_kernel(n_pad, nchunk):
    mesh = plsc.VectorSubcoreMesh(core_axis_name="c", subcore_axis_name="s")

    @functools.partial(
        pl.kernel,
        mesh=mesh,
        compiler_params=_sc_compiler_params(),
        out_type=jax.ShapeDtypeStruct((NW, nchunk, C), jnp.float32),
        scratch_types=[
            pltpu.VMEM((n_pad,), jnp.float32),
            pltpu.VMEM((n_pad,), jnp.float32),
            pltpu.VMEM((nchunk, C), jnp.int32),
            pltpu.VMEM((nchunk, C), jnp.int32),
            pltpu.VMEM((nchunk, C), jnp.float32),
        ],
    )
    def sc_weights(asrc_hbm, adst_hbm, src_hbm, dst_hbm, w_hbm,
                   asrc_v, adst_v, sidx_v, didx_v, w_all_v):
        c = lax.axis_index("c")
        s = lax.axis_index("s")
        wid = c * NS + s
        pltpu.sync_copy(src_hbm.at[wid], sidx_v)
        pltpu.sync_copy(dst_hbm.at[wid], didx_v)
        pltpu.sync_copy(asrc_hbm, asrc_v)
        pltpu.sync_copy(adst_hbm, adst_v)

        @pl.loop(0, nchunk)
        def _(i):
            @pl.loop(0, C, step=16)
            def _(j):
                s16 = sidx_v[i, pl.ds(j, 16)]
                d16 = didx_v[i, pl.ds(j, 16)]
                al = (plsc.load_gather(asrc_v, [s16])
                      + plsc.load_gather(adst_v, [d16]))
                al = jnp.where(al >= 0.0, al, al * 0.2)
                w_all_v[i, pl.ds(j, 16)] = jnp.exp(al)

        pltpu.sync_copy(w_all_v, w_hbm.at[wid])

    return sc_weights


def _sc_scatter_kernel(n_pad, nchunk):
    stripe = n_pad // NS
    mesh = plsc.VectorSubcoreMesh(core_axis_name="c", subcore_axis_name="s")

    @functools.partial(
        pl.kernel,
        mesh=mesh,
        compiler_params=_sc_compiler_params(),
        out_type=[
            jax.ShapeDtypeStruct((NC, n_pad, D), jnp.float32),
            jax.ShapeDtypeStruct((NW, n_pad), jnp.float32),
        ],
        scratch_types=[
            pltpu.VMEM((C,), jnp.int32),
            pltpu.VMEM((C,), jnp.float32),
            pltpu.VMEM((C,), jnp.int32),
            pltpu.VMEM((C, D), jnp.float32),
            pltpu.VMEM((n_pad,), jnp.float32),
            pltpu.VMEM_SHARED((n_pad, D), jnp.float32),
            pltpu.SemaphoreType.DMA,
        ],
    )
    def sc_scatter(x_hbm, src_hbm, dst_hbm, w_hbm,
                   acc_hbm, den_hbm,
                   didx_c, w_c, sidx_c, rows_v, den_v,
                   acc_sh, sem):
        c = lax.axis_index("c")
        s = lax.axis_index("s")
        wid = c * NS + s
        zero16 = jnp.zeros((16,), jnp.float32)
        iota16 = lax.iota(jnp.int32, 16)

        @pl.loop(0, C)
        def _(r):
            for k in range(D // 16):
                rows_v[r, pl.ds(k * 16, 16)] = zero16

        @pl.loop(0, n_pad, step=16)
        def _(i):
            den_v[pl.ds(i, 16)] = zero16

        for b in range(stripe // C):
            for j in range(C // 16):
                sidx_c[pl.ds(j * 16, 16)] = (
                    s * stripe + b * C + j * 16 + iota16)
            pltpu.sync_copy(rows_v, acc_sh.at[sidx_c])
        plsc.subcore_barrier()

        @pl.loop(0, nchunk)
        def _(i):
            pltpu.sync_copy(src_hbm.at[wid, i], sidx_c)
            gat = pltpu.async_copy(x_hbm.at[sidx_c], rows_v, sem)
            pltpu.sync_copy(dst_hbm.at[wid, i], didx_c)
            pltpu.sync_copy(w_hbm.at[wid, i], w_c)

            @pl.loop(0, C, step=16)
            def _(j):
                plsc.addupdate_scatter(den_v, [didx_c[pl.ds(j, 16)]],
                                       w_c[pl.ds(j, 16)])
            gat.wait()

            @pl.loop(0, C, step=4)
            def _(r0):
                for drr in range(4):
                    r = r0 + drr
                    bc = plsc.load_gather(w_c, [jnp.zeros((16,), jnp.int32) + r])
                    for k in range(D // 16):
                        rows_v[r, pl.ds(k * 16, 16)] = (
                            rows_v[r, pl.ds(k * 16, 16)] * bc)

            pltpu.sync_copy(rows_v, acc_sh.at[didx_c], add=True)

        plsc.subcore_barrier()

        for b in range(stripe // C):
            for j in range(C // 16):
                sidx_c[pl.ds(j * 16, 16)] = (
                    s * stripe + b * C + j * 16 + iota16)
            pltpu.sync_copy(acc_sh.at[sidx_c], rows_v)
            pltpu.sync_copy(rows_v,
                            acc_hbm.at[c, pl.ds(s * stripe + b * C, C)])
        pltpu.sync_copy(den_v, den_hbm.at[wid])

    return sc_scatter


def _fin_body(acc_ref, den_ref, x_ref, asrc_ref, adst_ref, b_ref, out_ref):
    num = acc_ref[0] + acc_ref[1]
    dtot = lax.dot_general(den_ref[...], jnp.ones((NW, 1), jnp.float32),
                           dimension_numbers=(((0,), (0,)), ((), ())),
                           precision=lax.Precision.HIGHEST,
                           preferred_element_type=jnp.float32)
    a = asrc_ref[...] + adst_ref[...]
    a = jnp.where(a >= 0.0, a, a * 0.2)
    ws = jnp.exp(a)
    num = num + ws * x_ref[...]
    out_ref[...] = num / (dtot + ws + 1e-16) + b_ref[...]


def _tc_finalize(acc, den, x, asrc, adst, bias2d, n_pad):
    return pl.pallas_call(
        _fin_body,
        grid=(n_pad // BN,),
        in_specs=[
            pl.BlockSpec((NC, BN, D), lambda i: (0, i, 0)),
            pl.BlockSpec((NW, BN), lambda i: (0, i)),
            pl.BlockSpec((BN, D), lambda i: (i, 0)),
            pl.BlockSpec((BN, 1), lambda i: (i, 0)),
            pl.BlockSpec((BN, 1), lambda i: (i, 0)),
            pl.BlockSpec((1, D), lambda i: (0, 0)),
        ],
        out_specs=pl.BlockSpec((BN, D), lambda i: (i, 0)),
        out_shape=jax.ShapeDtypeStruct((n_pad, D), jnp.float32),
    )(acc, den, x, asrc, adst, bias2d)


def kernel(feature, edge_index, W, att_src, att_dst, bias):
    N, d_in = feature.shape
    E = edge_index.shape[1]

    n_pad = ((N + 2047) // 2048) * 2048
    if n_pad == N:
        n_pad += 2048
    e_pad = ((E + NW * C - 1) // (NW * C)) * (NW * C)
    nchunk = e_pad // (NW * C)

    feature_p = jnp.concatenate(
        [feature, jnp.zeros((n_pad - N, d_in), jnp.float32)])
    src = jnp.concatenate([edge_index[0].astype(jnp.int32),
                           jnp.zeros((e_pad - E,), jnp.int32)])
    dst = jnp.concatenate([edge_index[1].astype(jnp.int32),
                           jnp.full((e_pad - E,), N, jnp.int32)])
    src = src.reshape(NW, nchunk, C)
    dst = dst.reshape(NW, nchunk, C)

    x, asrc, adst = _tc_prep(feature_p, W, att_src, att_dst, n_pad)

    w_edges = _sc_weights_kernel(n_pad, nchunk)(
        asrc.reshape(n_pad), adst.reshape(n_pad), src, dst)
    acc, den = _sc_scatter_kernel(n_pad, nchunk)(x, src, dst, w_edges)

    out = _tc_finalize(acc, den, x, asrc, adst, bias.reshape(1, D), n_pad)
    return out[:N]
